# baseline (speedup 1.0000x reference)
import functools
import jax
import jax.numpy as jnp
from jax import lax
from jax.experimental import pallas as pl
from jax.experimental.pallas import tpu as pltpu
from jax.experimental.pallas import tpu_sc as plsc

_NC = 2
_NS = 16
_NW = _NC * _NS

_BLK = 256
_CH = 32
_NBUF = 3


def _make_gather(D, M, name):
    bpw = M // _NW
    nch = bpw // _CH
    nbuf = min(_NBUF, nch)
    mesh = plsc.VectorSubcoreMesh(core_axis_name="c", subcore_axis_name="s")

    def body(table_hbm, idx_hbm, out_hbm, idx_v, rows_v, *sems):
        gs, ws = sems[:nbuf], sems[nbuf:]
        wid = lax.axis_index("s") * _NC + lax.axis_index("c")
        base0 = wid * bpw
        for c in range(nch):
            pltpu.sync_copy(idx_hbm.at[pl.ds(base0 + c * _CH, _CH)],
                            idx_v.at[c])
        gops = [None] * nbuf
        wops = [None] * nbuf
        finals = []
        for c in range(nbuf):
            gops[c] = pltpu.async_copy(table_hbm.at[idx_v.at[c]],
                                       rows_v.at[c], gs[c])
        for c in range(nch):
            b = c % nbuf
            gops[b].wait()
            wops[b] = pltpu.async_copy(
                rows_v.at[b], out_hbm.at[pl.ds(base0 + c * _CH, _CH)], ws[b])
            nxt = c + nbuf
            if nxt < nch:
                wops[b].wait()
                gops[b] = pltpu.async_copy(table_hbm.at[idx_v.at[nxt]],
                                           rows_v.at[b], gs[b])
            else:
                finals.append(wops[b])
        for wb in finals:
            wb.wait()

    body.__name__ = name
    return pl.kernel(
        body, mesh=mesh,
        out_type=jax.ShapeDtypeStruct((M, D), jnp.float32),
        scratch_types=(
            [pltpu.VMEM((nch, _CH), jnp.int32),
             pltpu.VMEM((nbuf, _CH, D), jnp.float32)]
            + [pltpu.SemaphoreType.DMA] * (2 * nbuf)
        ),
    )


def _expert_kernel(be_ref, xg_ref, gate_ref, logits_ref, masks_ref,
                   W1_ref, b1_ref, W2_ref, b2_ref,
                   yo_ref, guide_ref, *, n_views, n_experts, nb):
    s = pl.program_id(0)
    last = n_views * nb - 1

    @pl.when(s == 0)
    def _init():
        guide_ref[...] = jnp.zeros_like(guide_ref)

    @pl.when(lax.rem(s, nb) == 0)
    def _guide():
        logits = logits_ref[0]
        mask = masks_ref[0]
        probs = jax.nn.softmax(logits, axis=-1)
        imp = jnp.mean(probs, axis=0, keepdims=True)
        load = jnp.mean(mask, axis=0, keepdims=True)
        guide_ref[...] += n_experts * jnp.sum(imp * load)

    xb = xg_ref[...].astype(jnp.bfloat16)
    h = jnp.dot(xb, W1_ref[0, 0].astype(jnp.bfloat16),
                preferred_element_type=jnp.float32)
    h = jax.nn.gelu(h + b1_ref[0])
    eo = jnp.dot(h.astype(jnp.bfloat16), W2_ref[0, 0].astype(jnp.bfloat16),
                 preferred_element_type=jnp.float32)
    eo = eo + b2_ref[0]
    yo_ref[...] = gate_ref[...] * eo

    @pl.when(s == last)
    def _fin():
        guide_ref[...] = guide_ref[...] / n_views


def _finish_kernel(part_ref, x_ref, Wg1_ref, bg1_ref, Wg2_ref, bg2_ref,
                   gamma_ref, beta_ref, out_ref):
    x = x_ref[...]
    moe = ((part_ref[0] + part_ref[1]) + (part_ref[2] + part_ref[3]))
    gh = jnp.dot(x.astype(jnp.bfloat16), Wg1_ref[...].astype(jnp.bfloat16),
                 preferred_element_type=jnp.float32)
    gh = jax.nn.gelu(gh + bg1_ref[0])
    gen = jnp.dot(gh.astype(jnp.bfloat16), Wg2_ref[...].astype(jnp.bfloat16),
                  preferred_element_type=jnp.float32)
    y = moe + gen + bg2_ref[0] + x
    mu = jnp.mean(y, axis=-1, keepdims=True)
    var = jnp.mean(jnp.square(y - mu), axis=-1, keepdims=True)
    out_ref[...] = (y - mu) * lax.rsqrt(var + 1e-5) * gamma_ref[0] + beta_ref[0]


def kernel(x, total_logits, total_masks, W1, b1, W2, b2, Wg1, bg1, Wg2, bg2, gamma, beta):
    N, D = x.shape
    V, _, E = total_logits.shape
    F = W1.shape[-1]
    K = 2
    NK = N * K
    B = _BLK
    P = NK + E * B
    NB = P // B
    P_tot = V * P

    probs = jax.nn.softmax(total_logits, axis=-1)
    gated = probs * total_masks
    gated = gated / (jnp.sum(gated, axis=-1, keepdims=True) + 1e-9)
    _, topi = lax.top_k(total_masks, K)
    gate_pair = jnp.take_along_axis(gated, topi, axis=-1)
    ef = topi.reshape(V, NK).astype(jnp.int32)
    tokf = jnp.broadcast_to(
        jnp.arange(N, dtype=jnp.int32)[:, None], (N, K)).reshape(NK)
    oh = (ef[..., None] == jnp.arange(E, dtype=jnp.int32)).astype(jnp.int32)
    rank = jnp.sum((jnp.cumsum(oh, axis=1) - oh) * oh, axis=-1)
    counts = jnp.sum(oh, axis=1)
    padded = ((counts + B - 1) // B) * B
    starts = jnp.cumsum(padded, axis=1) - padded
    dest = jnp.take_along_axis(starts, ef, axis=1) + rank
    flat_dest = (dest + (jnp.arange(V, dtype=jnp.int32) * P)[:, None]).reshape(-1)
    tok_idx = jnp.zeros((P_tot,), jnp.int32).at[flat_dest].set(
        jnp.broadcast_to(tokf, (V, NK)).reshape(-1))
    gate_row = jnp.zeros((P_tot,), jnp.float32).at[flat_dest].set(
        gate_pair.reshape(-1))
    bstart = jnp.arange(NB, dtype=jnp.int32) * B
    be = jnp.sum(bstart[None, :, None] >= starts[:, None, :], axis=-1) - 1
    be = jnp.clip(be, 0, E - 1).astype(jnp.int32).reshape(-1)

    xg = _make_gather(D, P_tot, "route_gather")(x, tok_idx)

    b1r = b1.reshape(V * E, 1, F)
    b2r = b2.reshape(V * E, 1, D)
    grid_spec = pltpu.PrefetchScalarGridSpec(
        num_scalar_prefetch=1,
        grid=(V * NB,),
        in_specs=[
            pl.BlockSpec((B, D), lambda s, be_r: (s, 0)),
            pl.BlockSpec((B, 1), lambda s, be_r: (s, 0)),
            pl.BlockSpec((1, N, E), lambda s, be_r: (s // NB, 0, 0)),
            pl.BlockSpec((1, N, E), lambda s, be_r: (s // NB, 0, 0)),
            pl.BlockSpec((1, 1, D, F),
                         lambda s, be_r: (s // NB, be_r[s], 0, 0)),
            pl.BlockSpec((1, 1, F),
                         lambda s, be_r: ((s // NB) * E + be_r[s], 0, 0)),
            pl.BlockSpec((1, 1, F, D),
                         lambda s, be_r: (s // NB, be_r[s], 0, 0)),
            pl.BlockSpec((1, 1, D),
                         lambda s, be_r: ((s // NB) * E + be_r[s], 0, 0)),
        ],
        out_specs=[
            pl.BlockSpec((B, D), lambda s, be_r: (s, 0)),
            pl.BlockSpec((1, 1), lambda s, be_r: (0, 0)),
        ],
    )
    yo, guide = pl.pallas_call(
        functools.partial(_expert_kernel, n_views=V, n_experts=E, nb=NB),
        grid_spec=grid_spec,
        out_shape=[
            jax.ShapeDtypeStruct((P_tot, D), jnp.float32),
            jax.ShapeDtypeStruct((1, 1), jnp.float32),
        ],
        compiler_params=pltpu.CompilerParams(
            dimension_semantics=("arbitrary",),
        ),
    )(be, xg, gate_row.reshape(P_tot, 1), total_logits, total_masks,
      W1, b1r, W2, b2r)

    gidx = (dest.reshape(V, N, K).transpose(0, 2, 1)
            + (jnp.arange(V, dtype=jnp.int32) * P)[:, None, None]).reshape(-1)
    partial = _make_gather(D, V * K * N, "combine_gather")(yo, gidx).reshape(V * K, N, D)

    NT = 2
    Nc = N // NT
    out = pl.pallas_call(
        _finish_kernel,
        grid=(NT,),
        in_specs=[
            pl.BlockSpec((V * K, Nc, D), lambda t: (0, t, 0)),
            pl.BlockSpec((Nc, D), lambda t: (t, 0)),
            pl.BlockSpec((D, F), lambda t: (0, 0)),
            pl.BlockSpec((1, F), lambda t: (0, 0)),
            pl.BlockSpec((F, D), lambda t: (0, 0)),
            pl.BlockSpec((1, D), lambda t: (0, 0)),
            pl.BlockSpec((1, D), lambda t: (0, 0)),
            pl.BlockSpec((1, D), lambda t: (0, 0)),
        ],
        out_specs=pl.BlockSpec((Nc, D), lambda t: (t, 0)),
        out_shape=jax.ShapeDtypeStruct((N, D), jnp.float32),
        compiler_params=pltpu.CompilerParams(
            dimension_semantics=("arbitrary",),
        ),
    )(partial, x, Wg1, bg1.reshape(1, F), Wg2, bg2.reshape(1, D),
      gamma.reshape(1, D), beta.reshape(1, D))
    return out, guide[0, 0]


# speedup vs baseline: 1.5323x; 1.5323x over previous
import functools
import jax
import jax.numpy as jnp
from jax import lax
from jax.experimental import pallas as pl
from jax.experimental.pallas import tpu as pltpu
from jax.experimental.pallas import tpu_sc as plsc

_NC = 2
_NS = 16
_NW = _NC * _NS

_BLK = 256
_CH = 32
_NBUF = 3


def _make_gather(D, M, name):
    bpw = M // _NW
    nch = bpw // _CH
    nbuf = min(_NBUF, nch)
    mesh = plsc.VectorSubcoreMesh(core_axis_name="c", subcore_axis_name="s")

    def body(table_hbm, idx_hbm, out_hbm, idx_v, rows_v, *sems):
        gs, ws = sems[:nbuf], sems[nbuf:]
        wid = lax.axis_index("s") * _NC + lax.axis_index("c")
        base0 = wid * bpw
        for c in range(nch):
            pltpu.sync_copy(idx_hbm.at[pl.ds(base0 + c * _CH, _CH)],
                            idx_v.at[c])
        gops = [None] * nbuf
        wops = [None] * nbuf
        finals = []
        for c in range(nbuf):
            gops[c] = pltpu.async_copy(table_hbm.at[idx_v.at[c]],
                                       rows_v.at[c], gs[c])
        for c in range(nch):
            b = c % nbuf
            gops[b].wait()
            wops[b] = pltpu.async_copy(
                rows_v.at[b], out_hbm.at[pl.ds(base0 + c * _CH, _CH)], ws[b])
            nxt = c + nbuf
            if nxt < nch:
                wops[b].wait()
                gops[b] = pltpu.async_copy(table_hbm.at[idx_v.at[nxt]],
                                           rows_v.at[b], gs[b])
            else:
                finals.append(wops[b])
        for wb in finals:
            wb.wait()

    body.__name__ = name
    return pl.kernel(
        body, mesh=mesh,
        out_type=jax.ShapeDtypeStruct((M, D), jnp.float32),
        scratch_types=(
            [pltpu.VMEM((nch, _CH), jnp.int32),
             pltpu.VMEM((nbuf, _CH, D), jnp.float32)]
            + [pltpu.SemaphoreType.DMA] * (2 * nbuf)
        ),
    )


def _expert_kernel(be_ref, xg_ref, gate_ref, logits_ref, masks_ref,
                   W1_ref, b1_ref, W2_ref, b2_ref,
                   yo_ref, guide_ref, *, n_views, n_experts, nb):
    s = pl.program_id(0)
    last = n_views * nb - 1

    @pl.when(s == 0)
    def _init():
        guide_ref[...] = jnp.zeros_like(guide_ref)

    @pl.when(lax.rem(s, nb) == 0)
    def _guide():
        logits = logits_ref[0]
        mask = masks_ref[0]
        probs = jax.nn.softmax(logits, axis=-1)
        imp = jnp.mean(probs, axis=0, keepdims=True)
        load = jnp.mean(mask, axis=0, keepdims=True)
        guide_ref[...] += n_experts * jnp.sum(imp * load)

    xb = xg_ref[...].astype(jnp.bfloat16)
    h = jnp.dot(xb, W1_ref[0, 0].astype(jnp.bfloat16),
                preferred_element_type=jnp.float32)
    h = jax.nn.gelu(h + b1_ref[0])
    eo = jnp.dot(h.astype(jnp.bfloat16), W2_ref[0, 0].astype(jnp.bfloat16),
                 preferred_element_type=jnp.float32)
    eo = eo + b2_ref[0]
    yo_ref[...] = gate_ref[...] * eo

    @pl.when(s == last)
    def _fin():
        guide_ref[...] = guide_ref[...] / n_views


def _finish_kernel(part_ref, x_ref, Wg1_ref, bg1_ref, Wg2_ref, bg2_ref,
                   gamma_ref, beta_ref, out_ref):
    x = x_ref[...]
    moe = ((part_ref[0] + part_ref[1]) + (part_ref[2] + part_ref[3]))
    gh = jnp.dot(x.astype(jnp.bfloat16), Wg1_ref[...].astype(jnp.bfloat16),
                 preferred_element_type=jnp.float32)
    gh = jax.nn.gelu(gh + bg1_ref[0])
    gen = jnp.dot(gh.astype(jnp.bfloat16), Wg2_ref[...].astype(jnp.bfloat16),
                  preferred_element_type=jnp.float32)
    y = moe + gen + bg2_ref[0] + x
    mu = jnp.mean(y, axis=-1, keepdims=True)
    var = jnp.mean(jnp.square(y - mu), axis=-1, keepdims=True)
    out_ref[...] = (y - mu) * lax.rsqrt(var + 1e-5) * gamma_ref[0] + beta_ref[0]


def kernel(x, total_logits, total_masks, W1, b1, W2, b2, Wg1, bg1, Wg2, bg2, gamma, beta):
    N, D = x.shape
    V, _, E = total_logits.shape
    F = W1.shape[-1]
    K = 2
    NK = N * K
    B = _BLK
    P = NK + E * B
    NB = P // B
    P_tot = V * P

    probs = jax.nn.softmax(total_logits, axis=-1)
    gated = probs * total_masks
    gated = gated / (jnp.sum(gated, axis=-1, keepdims=True) + 1e-9)
    _, topi = lax.top_k(total_masks, K)
    gate_pair = jnp.take_along_axis(gated, topi, axis=-1)
    ef = topi.reshape(V, NK).astype(jnp.int32)
    tokf = jnp.broadcast_to(
        jnp.arange(N, dtype=jnp.int32)[:, None], (N, K)).reshape(NK)
    oh = (ef[..., None] == jnp.arange(E, dtype=jnp.int32)).astype(jnp.int32)
    rank = jnp.sum((jnp.cumsum(oh, axis=1) - oh) * oh, axis=-1)
    counts = jnp.sum(oh, axis=1)
    padded = ((counts + B - 1) // B) * B
    starts = jnp.cumsum(padded, axis=1) - padded
    dest = jnp.take_along_axis(starts, ef, axis=1) + rank
    flat_dest = (dest + (jnp.arange(V, dtype=jnp.int32) * P)[:, None]).reshape(-1)
    tok_idx = (jnp.arange(P_tot, dtype=jnp.int32) % N).at[flat_dest].set(
        jnp.broadcast_to(tokf, (V, NK)).reshape(-1))
    gate_row = jnp.zeros((P_tot,), jnp.float32).at[flat_dest].set(
        gate_pair.reshape(-1))
    bstart = jnp.arange(NB, dtype=jnp.int32) * B
    be = jnp.sum(bstart[None, :, None] >= starts[:, None, :], axis=-1) - 1
    be = jnp.clip(be, 0, E - 1).astype(jnp.int32).reshape(-1)

    xg = _make_gather(D, P_tot, "route_gather")(x, tok_idx)

    b1r = b1.reshape(V * E, 1, F)
    b2r = b2.reshape(V * E, 1, D)
    grid_spec = pltpu.PrefetchScalarGridSpec(
        num_scalar_prefetch=1,
        grid=(V * NB,),
        in_specs=[
            pl.BlockSpec((B, D), lambda s, be_r: (s, 0)),
            pl.BlockSpec((B, 1), lambda s, be_r: (s, 0)),
            pl.BlockSpec((1, N, E), lambda s, be_r: (s // NB, 0, 0)),
            pl.BlockSpec((1, N, E), lambda s, be_r: (s // NB, 0, 0)),
            pl.BlockSpec((1, 1, D, F),
                         lambda s, be_r: (s // NB, be_r[s], 0, 0)),
            pl.BlockSpec((1, 1, F),
                         lambda s, be_r: ((s // NB) * E + be_r[s], 0, 0)),
            pl.BlockSpec((1, 1, F, D),
                         lambda s, be_r: (s // NB, be_r[s], 0, 0)),
            pl.BlockSpec((1, 1, D),
                         lambda s, be_r: ((s // NB) * E + be_r[s], 0, 0)),
        ],
        out_specs=[
            pl.BlockSpec((B, D), lambda s, be_r: (s, 0)),
            pl.BlockSpec((1, 1), lambda s, be_r: (0, 0)),
        ],
    )
    yo, guide = pl.pallas_call(
        functools.partial(_expert_kernel, n_views=V, n_experts=E, nb=NB),
        grid_spec=grid_spec,
        out_shape=[
            jax.ShapeDtypeStruct((P_tot, D), jnp.float32),
            jax.ShapeDtypeStruct((1, 1), jnp.float32),
        ],
        compiler_params=pltpu.CompilerParams(
            dimension_semantics=("arbitrary",),
        ),
    )(be, xg, gate_row.reshape(P_tot, 1), total_logits, total_masks,
      W1, b1r, W2, b2r)

    gidx = (dest.reshape(V, N, K).transpose(0, 2, 1)
            + (jnp.arange(V, dtype=jnp.int32) * P)[:, None, None]).reshape(-1)
    partial = _make_gather(D, V * K * N, "combine_gather")(yo, gidx).reshape(V * K, N, D)

    NT = 2
    Nc = N // NT
    out = pl.pallas_call(
        _finish_kernel,
        grid=(NT,),
        in_specs=[
            pl.BlockSpec((V * K, Nc, D), lambda t: (0, t, 0)),
            pl.BlockSpec((Nc, D), lambda t: (t, 0)),
            pl.BlockSpec((D, F), lambda t: (0, 0)),
            pl.BlockSpec((1, F), lambda t: (0, 0)),
            pl.BlockSpec((F, D), lambda t: (0, 0)),
            pl.BlockSpec((1, D), lambda t: (0, 0)),
            pl.BlockSpec((1, D), lambda t: (0, 0)),
            pl.BlockSpec((1, D), lambda t: (0, 0)),
        ],
        out_specs=pl.BlockSpec((Nc, D), lambda t: (t, 0)),
        out_shape=jax.ShapeDtypeStruct((N, D), jnp.float32),
        compiler_params=pltpu.CompilerParams(
            dimension_semantics=("arbitrary",),
        ),
    )(partial, x, Wg1, bg1.reshape(1, F), Wg2, bg2.reshape(1, D),
      gamma.reshape(1, D), beta.reshape(1, D))
    return out, guide[0, 0]
